# baseline (speedup 1.0000x reference)
import functools

import jax
import jax.numpy as jnp
import numpy as np
from jax import lax
from jax.experimental import pallas as pl
from jax.experimental.pallas import tpu as pltpu
from jax.experimental.pallas import tpu_sc as plsc

N_ROWS = 65536
N_COLS = 64
MAGIC = np.float32(12582912.0)

NC = 2
NS = 16
L = 16
NW = NC * NS
ROWS_PER_W = N_ROWS // NW
CHUNK = 256
CELEMS = CHUNK * N_COLS
N_CHUNKS = ROWS_PER_W // CHUNK
BLOCKS = CHUNK // L
PITCH = 17

_mesh = plsc.VectorSubcoreMesh(core_axis_name="c", subcore_axis_name="s")


@functools.partial(
    pl.kernel,
    mesh=_mesh,
    out_type=jax.ShapeDtypeStruct((N_ROWS * N_COLS,), jnp.float32),
    scratch_types=[
        pltpu.VMEM((CELEMS,), jnp.float32),
        pltpu.VMEM((CELEMS,), jnp.float32),
        pltpu.VMEM((CELEMS,), jnp.float32),
        pltpu.VMEM((CELEMS,), jnp.float32),
        pltpu.VMEM((CHUNK * PITCH,), jnp.float32),
        pltpu.VMEM((CHUNK * PITCH,), jnp.int32),
        pltpu.VMEM((CHUNK * PITCH,), jnp.float32),
        pltpu.SemaphoreType.DMA,
        pltpu.SemaphoreType.DMA,
        pltpu.SemaphoreType.DMA,
        pltpu.SemaphoreType.DMA,
    ],
    compiler_params=pltpu.CompilerParams(needs_layout_passes=False),
)
def _dn_quantize(x_hbm, out_hbm, in0, in1, ou0, ou1, sa, sk, ssum,
                 si0, si1, so0, so1):
    wid = lax.axis_index("s") * NC + lax.axis_index("c")
    iota = lax.iota(jnp.int32, L)
    iota17 = iota * PITCH
    iota64 = iota * N_COLS
    kcol = [iota + g * L for g in range(4)]
    w_elem0 = wid * (ROWS_PER_W * N_COLS)

    def in_slice(t):
        return x_hbm.at[pl.ds(w_elem0 + t * CELEMS, CELEMS)]

    def out_slice(t):
        return out_hbm.at[pl.ds(w_elem0 + t * CELEMS, CELEMS)]

    def compute_chunk(in_buf, out_buf):
        @plsc.parallel_loop(0, CHUNK, unroll=8)
        def _row(r):
            b64 = r * N_COLS
            v = [in_buf[pl.ds(b64 + g * L, L)] for g in range(4)]
            f = [(vg + MAGIC) - MAGIC for vg in v]
            a = [jnp.abs(v[g] - f[g]) for g in range(4)]
            for g in range(4):
                out_buf[pl.ds(b64 + g * L, L)] = f[g]
            m01 = jnp.maximum(a[0], a[1])
            k01 = jnp.where(a[1] > a[0], kcol[1], kcol[0])
            m23 = jnp.maximum(a[2], a[3])
            k23 = jnp.where(a[3] > a[2], kcol[3], kcol[2])
            mm = jnp.maximum(m01, m23)
            kk = jnp.where(m23 > m01, k23, k01)
            s = (f[0] + f[1]) + (f[2] + f[3])
            r17 = r * PITCH
            sa[pl.ds(r17, L)] = mm
            sk[pl.ds(r17, L)] = kk
            ssum[pl.ds(r17, L)] = s

        @plsc.parallel_loop(0, BLOCKS, unroll=4)
        def _blk(b):
            base17 = iota17 + b * (L * PITCH)
            m = [jnp.full((L,), -1.0, jnp.float32)] * 4
            kb = [jnp.zeros((L,), jnp.int32)] * 4
            acc = [jnp.zeros((L,), jnp.float32)] * 4
            for j2 in range(L):
                c = j2 // 4
                aj = plsc.load_gather(sa, [base17 + j2])
                kj = plsc.load_gather(sk, [base17 + j2])
                sj = plsc.load_gather(ssum, [base17 + j2])
                p = (aj > m[c]) | ((aj == m[c]) & (kj < kb[c]))
                m[c] = jnp.where(p, aj, m[c])
                kb[c] = jnp.where(p, kj, kb[c])
                acc[c] = acc[c] + sj
            mm, kk, ss = m[0], kb[0], acc[0]
            for c in range(1, 4):
                p = (m[c] > mm) | ((m[c] == mm) & (kb[c] < kk))
                mm = jnp.where(p, m[c], mm)
                kk = jnp.where(p, kb[c], kk)
                ss = ss + acc[c]
            odd = (ss.astype(jnp.int32) & 1) == 1
            tgt = iota64 + b * (L * N_COLS) + kk
            vk = plsc.load_gather(in_buf, [tgt])
            fk = (vk + MAGIC) - MAGIC
            fix = jnp.where(vk - fk < 0, jnp.float32(-1.0), jnp.float32(1.0))
            plsc.addupdate_scatter(out_buf, [tgt], fix, mask=odd)

    def slot(u, t, in_buf, out_buf, in_sem, out_sem):
        @pl.when(u > 0)
        def _():
            pltpu.make_async_copy(out_buf, out_slice(t - 2), out_sem).wait()

        pltpu.make_async_copy(in_slice(t), in_buf, in_sem).wait()
        compute_chunk(in_buf, out_buf)
        pltpu.async_copy(out_buf, out_slice(t), out_sem)

        @pl.when(u < N_CHUNKS // 2 - 1)
        def _():
            pltpu.async_copy(in_slice(t + 2), in_buf, in_sem)

    pltpu.async_copy(in_slice(0), in0, si0)
    pltpu.async_copy(in_slice(1), in1, si1)

    def pair_body(u, carry):
        slot(u, 2 * u, in0, ou0, si0, so0)
        slot(u, 2 * u + 1, in1, ou1, si1, so1)
        return carry

    lax.fori_loop(0, N_CHUNKS // 2, pair_body, 0)

    last = N_CHUNKS - 2
    pltpu.make_async_copy(ou0, out_slice(last), so0).wait()
    pltpu.make_async_copy(ou1, out_slice(last + 1), so1).wait()


def kernel(x):
    return _dn_quantize(x.reshape(N_ROWS * N_COLS)).reshape(N_ROWS, N_COLS)


# speedup vs baseline: 1.0180x; 1.0180x over previous
import functools

import jax
import jax.numpy as jnp
import numpy as np
from jax import lax
from jax.experimental import pallas as pl
from jax.experimental.pallas import tpu as pltpu
from jax.experimental.pallas import tpu_sc as plsc

N_ROWS = 65536
N_COLS = 64
MAGIC = np.float32(12582912.0)

NC = 2
NS = 16
L = 16
NW = NC * NS
ROWS_PER_W = N_ROWS // NW
CHUNK = 256
CELEMS = CHUNK * N_COLS
N_CHUNKS = ROWS_PER_W // CHUNK
BLOCKS = CHUNK // L
PITCH = 17

_mesh = plsc.VectorSubcoreMesh(core_axis_name="c", subcore_axis_name="s")


@functools.partial(
    pl.kernel,
    mesh=_mesh,
    out_type=jax.ShapeDtypeStruct((N_ROWS * N_COLS,), jnp.float32),
    scratch_types=[
        pltpu.VMEM((CELEMS,), jnp.float32),
        pltpu.VMEM((CELEMS,), jnp.float32),
        pltpu.VMEM((CELEMS,), jnp.float32),
        pltpu.VMEM((CELEMS,), jnp.float32),
        pltpu.VMEM((CHUNK * PITCH,), jnp.float32),
        pltpu.VMEM((CHUNK * PITCH,), jnp.int32),
        pltpu.VMEM((CHUNK * PITCH,), jnp.float32),
        pltpu.SemaphoreType.DMA,
        pltpu.SemaphoreType.DMA,
        pltpu.SemaphoreType.DMA,
        pltpu.SemaphoreType.DMA,
    ],
    compiler_params=pltpu.CompilerParams(needs_layout_passes=False),
)
def _dn_quantize(x_hbm, out_hbm, in0, in1, ou0, ou1, sa, sk, ssum,
                 si0, si1, so0, so1):
    wid = lax.axis_index("s") * NC + lax.axis_index("c")
    iota = lax.iota(jnp.int32, L)
    iota17 = iota * PITCH
    iota64 = iota * N_COLS
    kcol = [iota + g * L for g in range(4)]
    w_elem0 = wid * (ROWS_PER_W * N_COLS)

    def in_slice(t):
        return x_hbm.at[pl.ds(w_elem0 + t * CELEMS, CELEMS)]

    def out_slice(t):
        return out_hbm.at[pl.ds(w_elem0 + t * CELEMS, CELEMS)]

    def compute_chunk(in_buf, out_buf):
        @plsc.parallel_loop(0, CHUNK, unroll=4)
        def _row(r):
            b64 = r * N_COLS
            v = [in_buf[pl.ds(b64 + g * L, L)] for g in range(4)]
            f = [(vg + MAGIC) - MAGIC for vg in v]
            a = [jnp.abs(v[g] - f[g]) for g in range(4)]
            for g in range(4):
                out_buf[pl.ds(b64 + g * L, L)] = f[g]
            m01 = jnp.maximum(a[0], a[1])
            k01 = jnp.where(a[1] > a[0], kcol[1], kcol[0])
            m23 = jnp.maximum(a[2], a[3])
            k23 = jnp.where(a[3] > a[2], kcol[3], kcol[2])
            mm = jnp.maximum(m01, m23)
            kk = jnp.where(m23 > m01, k23, k01)
            s = (f[0] + f[1]) + (f[2] + f[3])
            r17 = r * PITCH
            sa[pl.ds(r17, L)] = mm
            sk[pl.ds(r17, L)] = kk
            ssum[pl.ds(r17, L)] = s

        @plsc.parallel_loop(0, BLOCKS, unroll=4)
        def _blk(b):
            base17 = iota17 + b * (L * PITCH)
            m = [jnp.full((L,), -1.0, jnp.float32)] * 4
            kb = [jnp.zeros((L,), jnp.int32)] * 4
            acc = [jnp.zeros((L,), jnp.float32)] * 4
            for j2 in range(L):
                c = j2 // 4
                aj = plsc.load_gather(sa, [base17 + j2])
                kj = plsc.load_gather(sk, [base17 + j2])
                sj = plsc.load_gather(ssum, [base17 + j2])
                p = (aj > m[c]) | ((aj == m[c]) & (kj < kb[c]))
                m[c] = jnp.where(p, aj, m[c])
                kb[c] = jnp.where(p, kj, kb[c])
                acc[c] = acc[c] + sj
            mm, kk, ss = m[0], kb[0], acc[0]
            for c in range(1, 4):
                p = (m[c] > mm) | ((m[c] == mm) & (kb[c] < kk))
                mm = jnp.where(p, m[c], mm)
                kk = jnp.where(p, kb[c], kk)
                ss = ss + acc[c]
            odd = (ss.astype(jnp.int32) & 1) == 1
            tgt = iota64 + b * (L * N_COLS) + kk
            vk = plsc.load_gather(in_buf, [tgt])
            fk = (vk + MAGIC) - MAGIC
            fix = jnp.where(vk - fk < 0, jnp.float32(-1.0), jnp.float32(1.0))
            plsc.addupdate_scatter(out_buf, [tgt], fix, mask=odd)

    def slot(u, t, in_buf, out_buf, in_sem, out_sem):
        @pl.when(u > 0)
        def _():
            pltpu.make_async_copy(out_buf, out_slice(t - 2), out_sem).wait()

        pltpu.make_async_copy(in_slice(t), in_buf, in_sem).wait()
        compute_chunk(in_buf, out_buf)
        pltpu.async_copy(out_buf, out_slice(t), out_sem)

        @pl.when(u < N_CHUNKS // 2 - 1)
        def _():
            pltpu.async_copy(in_slice(t + 2), in_buf, in_sem)

    pltpu.async_copy(in_slice(0), in0, si0)
    pltpu.async_copy(in_slice(1), in1, si1)

    def pair_body(u, carry):
        slot(u, 2 * u, in0, ou0, si0, so0)
        slot(u, 2 * u + 1, in1, ou1, si1, so1)
        return carry

    lax.fori_loop(0, N_CHUNKS // 2, pair_body, 0)

    last = N_CHUNKS - 2
    pltpu.make_async_copy(ou0, out_slice(last), so0).wait()
    pltpu.make_async_copy(ou1, out_slice(last + 1), so1).wait()


def kernel(x):
    return _dn_quantize(x.reshape(N_ROWS * N_COLS)).reshape(N_ROWS, N_COLS)


# speedup vs baseline: 1.1181x; 1.0983x over previous
import functools

import jax
import jax.numpy as jnp
import numpy as np
from jax import lax
from jax.experimental import pallas as pl
from jax.experimental.pallas import tpu as pltpu
from jax.experimental.pallas import tpu_sc as plsc

N_ROWS = 65536
N_COLS = 64
MAGIC = np.float32(12582912.0)

NC = 2
NS = 16
L = 16
NW = NC * NS
ROWS_PER_W = N_ROWS // NW
CHUNK = 256
CELEMS = CHUNK * N_COLS
N_CHUNKS = ROWS_PER_W // CHUNK
BLOCKS = CHUNK // L
PITCH = 17

_mesh = plsc.VectorSubcoreMesh(core_axis_name="c", subcore_axis_name="s")


@functools.partial(
    pl.kernel,
    mesh=_mesh,
    out_type=jax.ShapeDtypeStruct((N_ROWS * N_COLS,), jnp.float32),
    scratch_types=[
        pltpu.VMEM((CELEMS,), jnp.float32),
        pltpu.VMEM((CELEMS,), jnp.float32),
        pltpu.VMEM((CELEMS,), jnp.float32),
        pltpu.VMEM((CELEMS,), jnp.float32),
        pltpu.VMEM((CHUNK * PITCH,), jnp.float32),
        pltpu.VMEM((CHUNK * PITCH,), jnp.int32),
        pltpu.VMEM((CHUNK * PITCH,), jnp.float32),
        pltpu.SemaphoreType.DMA,
        pltpu.SemaphoreType.DMA,
        pltpu.SemaphoreType.DMA,
        pltpu.SemaphoreType.DMA,
    ],
    compiler_params=pltpu.CompilerParams(needs_layout_passes=False),
)
def _dn_quantize(x_hbm, out_hbm, in0, in1, ou0, ou1, sa, sk, ssum,
                 si0, si1, so0, so1):
    wid = lax.axis_index("s") * NC + lax.axis_index("c")
    iota = lax.iota(jnp.int32, L)
    iota17 = iota * PITCH
    iota64 = iota * N_COLS
    kcol = [iota + g * L for g in range(4)]
    w_elem0 = wid * (ROWS_PER_W * N_COLS)

    def in_slice(t):
        return x_hbm.at[pl.ds(w_elem0 + t * CELEMS, CELEMS)]

    def out_slice(t):
        return out_hbm.at[pl.ds(w_elem0 + t * CELEMS, CELEMS)]

    def compute_chunk(in_buf, out_buf):
        @plsc.parallel_loop(0, CHUNK, unroll=4)
        def _row(r):
            b64 = r * N_COLS
            v = [in_buf[pl.ds(b64 + g * L, L)] for g in range(4)]
            f = [(vg + MAGIC) - MAGIC for vg in v]
            a = [jnp.abs(v[g] - f[g]) for g in range(4)]
            for g in range(4):
                out_buf[pl.ds(b64 + g * L, L)] = f[g]
            m01 = jnp.maximum(a[0], a[1])
            k01 = jnp.where(a[1] > a[0], kcol[1], kcol[0])
            m23 = jnp.maximum(a[2], a[3])
            k23 = jnp.where(a[3] > a[2], kcol[3], kcol[2])
            mm = jnp.maximum(m01, m23)
            kk = jnp.where(m23 > m01, k23, k01)
            s = (f[0] + f[1]) + (f[2] + f[3])
            r17 = r * PITCH
            sa[pl.ds(r17, L)] = mm
            sk[pl.ds(r17, L)] = kk
            ssum[pl.ds(r17, L)] = s

        @plsc.parallel_loop(0, BLOCKS, unroll=2)
        def _blk(b):
            base17 = iota17 + b * (L * PITCH)
            m = [jnp.full((L,), -1.0, jnp.float32)] * 2
            kb = [jnp.zeros((L,), jnp.int32)] * 2
            acc = [jnp.zeros((L,), jnp.float32)] * 2
            for j2 in range(L):
                c = j2 // 8
                aj = plsc.load_gather(sa, [base17 + j2])
                kj = plsc.load_gather(sk, [base17 + j2])
                sj = plsc.load_gather(ssum, [base17 + j2])
                p = (aj > m[c]) | ((aj == m[c]) & (kj < kb[c]))
                m[c] = jnp.where(p, aj, m[c])
                kb[c] = jnp.where(p, kj, kb[c])
                acc[c] = acc[c] + sj
            p = (m[1] > m[0]) | ((m[1] == m[0]) & (kb[1] < kb[0]))
            mm = jnp.where(p, m[1], m[0])
            kk = jnp.where(p, kb[1], kb[0])
            ss = acc[0] + acc[1]
            odd = (ss.astype(jnp.int32) & 1) == 1
            tgt = iota64 + b * (L * N_COLS) + kk
            vk = plsc.load_gather(in_buf, [tgt])
            fk = (vk + MAGIC) - MAGIC
            fix = jnp.where(vk - fk < 0, jnp.float32(-1.0), jnp.float32(1.0))
            plsc.addupdate_scatter(out_buf, [tgt], fix, mask=odd)

    def slot(u, t, in_buf, out_buf, in_sem, out_sem):
        @pl.when(u > 0)
        def _():
            pltpu.make_async_copy(out_buf, out_slice(t - 2), out_sem).wait()

        pltpu.make_async_copy(in_slice(t), in_buf, in_sem).wait()
        compute_chunk(in_buf, out_buf)
        pltpu.async_copy(out_buf, out_slice(t), out_sem)

        @pl.when(u < N_CHUNKS // 2 - 1)
        def _():
            pltpu.async_copy(in_slice(t + 2), in_buf, in_sem)

    pltpu.async_copy(in_slice(0), in0, si0)
    pltpu.async_copy(in_slice(1), in1, si1)

    def pair_body(u, carry):
        slot(u, 2 * u, in0, ou0, si0, so0)
        slot(u, 2 * u + 1, in1, ou1, si1, so1)
        return carry

    lax.fori_loop(0, N_CHUNKS // 2, pair_body, 0)

    last = N_CHUNKS - 2
    pltpu.make_async_copy(ou0, out_slice(last), so0).wait()
    pltpu.make_async_copy(ou1, out_slice(last + 1), so1).wait()


def kernel(x):
    return _dn_quantize(x.reshape(N_ROWS * N_COLS)).reshape(N_ROWS, N_COLS)


# speedup vs baseline: 1.4796x; 1.3233x over previous
import functools

import jax
import jax.numpy as jnp
import numpy as np
from jax import lax
from jax.experimental import pallas as pl
from jax.experimental.pallas import tpu as pltpu
from jax.experimental.pallas import tpu_sc as plsc

N_ROWS = 65536
N_COLS = 64
MAGIC = np.float32(12582912.0)

NC = 2
NS = 16
L = 16
NW = NC * NS
ROWS_PER_W = N_ROWS // NW
CHUNK = 128
N_CHUNKS = ROWS_PER_W // CHUNK
BLOCKS = CHUNK // L
PITCH = 17

_mesh = plsc.VectorSubcoreMesh(core_axis_name="c", subcore_axis_name="s")


@functools.partial(
    pl.kernel,
    mesh=_mesh,
    out_type=jax.ShapeDtypeStruct((N_ROWS, N_COLS), jnp.float32),
    scratch_types=[
        pltpu.VMEM((CHUNK, N_COLS), jnp.float32),
        pltpu.VMEM((CHUNK, N_COLS), jnp.float32),
        pltpu.VMEM((CHUNK, N_COLS), jnp.float32),
        pltpu.VMEM((CHUNK, N_COLS), jnp.float32),
        pltpu.VMEM((CHUNK * PITCH,), jnp.float32),
        pltpu.VMEM((CHUNK * PITCH,), jnp.int32),
        pltpu.VMEM((CHUNK * PITCH,), jnp.float32),
        pltpu.SemaphoreType.DMA,
        pltpu.SemaphoreType.DMA,
        pltpu.SemaphoreType.DMA,
        pltpu.SemaphoreType.DMA,
    ],
    compiler_params=pltpu.CompilerParams(needs_layout_passes=False),
)
def _dn_quantize(x_hbm, out_hbm, in0, in1, ou0, ou1, sa, sk, ssum,
                 si0, si1, so0, so1):
    wid = lax.axis_index("s") * NC + lax.axis_index("c")
    iota = lax.iota(jnp.int32, L)
    iota17 = iota * PITCH
    kcol = [iota + g * L for g in range(4)]
    w_row0 = wid * ROWS_PER_W

    def in_slice(t):
        return x_hbm.at[pl.ds(w_row0 + t * CHUNK, CHUNK)]

    def out_slice(t):
        return out_hbm.at[pl.ds(w_row0 + t * CHUNK, CHUNK)]

    def compute_chunk(in_buf, out_buf):
        @plsc.parallel_loop(0, CHUNK, unroll=4)
        def _row(r):
            v = [in_buf[r, pl.ds(g * L, L)] for g in range(4)]
            f = [(vg + MAGIC) - MAGIC for vg in v]
            a = [jnp.abs(v[g] - f[g]) for g in range(4)]
            for g in range(4):
                out_buf[r, pl.ds(g * L, L)] = f[g]
            m01 = jnp.maximum(a[0], a[1])
            k01 = jnp.where(a[1] > a[0], kcol[1], kcol[0])
            m23 = jnp.maximum(a[2], a[3])
            k23 = jnp.where(a[3] > a[2], kcol[3], kcol[2])
            mm = jnp.maximum(m01, m23)
            kk = jnp.where(m23 > m01, k23, k01)
            s = (f[0] + f[1]) + (f[2] + f[3])
            r17 = r * PITCH
            sa[pl.ds(r17, L)] = mm
            sk[pl.ds(r17, L)] = kk
            ssum[pl.ds(r17, L)] = s

        @plsc.parallel_loop(0, BLOCKS, unroll=2)
        def _blk(b):
            base17 = iota17 + b * (L * PITCH)
            m = [jnp.full((L,), -1.0, jnp.float32)] * 2
            kb = [jnp.zeros((L,), jnp.int32)] * 2
            acc = [jnp.zeros((L,), jnp.float32)] * 2
            for j2 in range(L):
                c = j2 // 8
                aj = plsc.load_gather(sa, [base17 + j2])
                kj = plsc.load_gather(sk, [base17 + j2])
                sj = plsc.load_gather(ssum, [base17 + j2])
                p = (aj > m[c]) | ((aj == m[c]) & (kj < kb[c]))
                m[c] = jnp.where(p, aj, m[c])
                kb[c] = jnp.where(p, kj, kb[c])
                acc[c] = acc[c] + sj
            p = (m[1] > m[0]) | ((m[1] == m[0]) & (kb[1] < kb[0]))
            mm = jnp.where(p, m[1], m[0])
            kk = jnp.where(p, kb[1], kb[0])
            ss = acc[0] + acc[1]
            odd = (ss.astype(jnp.int32) & 1) == 1
            rowv = iota + b * L
            vk = plsc.load_gather(in_buf, [rowv, kk])
            fk = (vk + MAGIC) - MAGIC
            fix = jnp.where(vk - fk < 0, jnp.float32(-1.0), jnp.float32(1.0))
            plsc.addupdate_scatter(out_buf, [rowv, kk], fix, mask=odd)

    def slot(u, t, in_buf, out_buf, in_sem, out_sem):
        @pl.when(u > 0)
        def _():
            pltpu.make_async_copy(out_buf, out_slice(t - 2), out_sem).wait()

        pltpu.make_async_copy(in_slice(t), in_buf, in_sem).wait()
        compute_chunk(in_buf, out_buf)
        pltpu.async_copy(out_buf, out_slice(t), out_sem)

        @pl.when(u < N_CHUNKS // 2 - 1)
        def _():
            pltpu.async_copy(in_slice(t + 2), in_buf, in_sem)

    pltpu.async_copy(in_slice(0), in0, si0)
    pltpu.async_copy(in_slice(1), in1, si1)

    def pair_body(u, carry):
        slot(u, 2 * u, in0, ou0, si0, so0)
        slot(u, 2 * u + 1, in1, ou1, si1, so1)
        return carry

    lax.fori_loop(0, N_CHUNKS // 2, pair_body, 0)

    last = N_CHUNKS - 2
    pltpu.make_async_copy(ou0, out_slice(last), so0).wait()
    pltpu.make_async_copy(ou1, out_slice(last + 1), so1).wait()


def kernel(x):
    return _dn_quantize(x)


# speedup vs baseline: 1.4828x; 1.0021x over previous
import functools

import jax
import jax.numpy as jnp
import numpy as np
from jax import lax
from jax.experimental import pallas as pl
from jax.experimental.pallas import tpu as pltpu
from jax.experimental.pallas import tpu_sc as plsc

N_ROWS = 65536
N_COLS = 64
MAGIC = np.float32(12582912.0)

NC = 2
NS = 16
L = 16
NW = NC * NS
ROWS_PER_W = N_ROWS // NW
CHUNK = 128
N_CHUNKS = ROWS_PER_W // CHUNK
BLOCKS = CHUNK // L
PITCH = 17

_mesh = plsc.VectorSubcoreMesh(core_axis_name="c", subcore_axis_name="s")


@functools.partial(
    pl.kernel,
    mesh=_mesh,
    out_type=jax.ShapeDtypeStruct((N_ROWS, N_COLS), jnp.float32),
    scratch_types=[
        pltpu.VMEM((CHUNK, N_COLS), jnp.float32),
        pltpu.VMEM((CHUNK, N_COLS), jnp.float32),
        pltpu.VMEM((CHUNK, N_COLS), jnp.float32),
        pltpu.VMEM((CHUNK, N_COLS), jnp.float32),
        pltpu.VMEM((CHUNK * PITCH,), jnp.float32),
        pltpu.VMEM((CHUNK * PITCH,), jnp.int32),
        pltpu.VMEM((CHUNK * PITCH,), jnp.float32),
        pltpu.SemaphoreType.DMA,
        pltpu.SemaphoreType.DMA,
        pltpu.SemaphoreType.DMA,
        pltpu.SemaphoreType.DMA,
    ],
    compiler_params=pltpu.CompilerParams(
        needs_layout_passes=False, use_tc_tiling_on_sc=True),
)
def _dn_quantize(x_hbm, out_hbm, in0, in1, ou0, ou1, sa, sk, ssum,
                 si0, si1, so0, so1):
    wid = lax.axis_index("s") * NC + lax.axis_index("c")
    iota = lax.iota(jnp.int32, L)
    iota17 = iota * PITCH
    kcol = [iota + g * L for g in range(4)]
    w_row0 = wid * ROWS_PER_W

    def in_slice(t):
        return x_hbm.at[pl.ds(w_row0 + t * CHUNK, CHUNK)]

    def out_slice(t):
        return out_hbm.at[pl.ds(w_row0 + t * CHUNK, CHUNK)]

    def compute_chunk(in_buf, out_buf):
        @plsc.parallel_loop(0, CHUNK, unroll=4)
        def _row(r):
            v = [in_buf[r, pl.ds(g * L, L)] for g in range(4)]
            f = [(vg + MAGIC) - MAGIC for vg in v]
            a = [jnp.abs(v[g] - f[g]) for g in range(4)]
            for g in range(4):
                out_buf[r, pl.ds(g * L, L)] = f[g]
            m01 = jnp.maximum(a[0], a[1])
            k01 = jnp.where(a[1] > a[0], kcol[1], kcol[0])
            m23 = jnp.maximum(a[2], a[3])
            k23 = jnp.where(a[3] > a[2], kcol[3], kcol[2])
            mm = jnp.maximum(m01, m23)
            kk = jnp.where(m23 > m01, k23, k01)
            s = (f[0] + f[1]) + (f[2] + f[3])
            r17 = r * PITCH
            sa[pl.ds(r17, L)] = mm
            sk[pl.ds(r17, L)] = kk
            ssum[pl.ds(r17, L)] = s

        @plsc.parallel_loop(0, BLOCKS, unroll=2)
        def _blk(b):
            base17 = iota17 + b * (L * PITCH)
            m = [jnp.full((L,), -1.0, jnp.float32)] * 2
            kb = [jnp.zeros((L,), jnp.int32)] * 2
            acc = [jnp.zeros((L,), jnp.float32)] * 2
            for j2 in range(L):
                c = j2 // 8
                aj = plsc.load_gather(sa, [base17 + j2])
                kj = plsc.load_gather(sk, [base17 + j2])
                sj = plsc.load_gather(ssum, [base17 + j2])
                p = (aj > m[c]) | ((aj == m[c]) & (kj < kb[c]))
                m[c] = jnp.where(p, aj, m[c])
                kb[c] = jnp.where(p, kj, kb[c])
                acc[c] = acc[c] + sj
            p = (m[1] > m[0]) | ((m[1] == m[0]) & (kb[1] < kb[0]))
            mm = jnp.where(p, m[1], m[0])
            kk = jnp.where(p, kb[1], kb[0])
            ss = acc[0] + acc[1]
            odd = (ss.astype(jnp.int32) & 1) == 1
            rowv = iota + b * L
            vk = plsc.load_gather(in_buf, [rowv, kk])
            fk = (vk + MAGIC) - MAGIC
            fix = jnp.where(vk - fk < 0, jnp.float32(-1.0), jnp.float32(1.0))
            plsc.addupdate_scatter(out_buf, [rowv, kk], fix, mask=odd)

    def slot(u, t, in_buf, out_buf, in_sem, out_sem):
        @pl.when(u > 0)
        def _():
            pltpu.make_async_copy(out_buf, out_slice(t - 2), out_sem).wait()

        pltpu.make_async_copy(in_slice(t), in_buf, in_sem).wait()
        compute_chunk(in_buf, out_buf)
        pltpu.async_copy(out_buf, out_slice(t), out_sem)

        @pl.when(u < N_CHUNKS // 2 - 1)
        def _():
            pltpu.async_copy(in_slice(t + 2), in_buf, in_sem)

    pltpu.async_copy(in_slice(0), in0, si0)
    pltpu.async_copy(in_slice(1), in1, si1)

    def pair_body(u, carry):
        slot(u, 2 * u, in0, ou0, si0, so0)
        slot(u, 2 * u + 1, in1, ou1, si1, so1)
        return carry

    lax.fori_loop(0, N_CHUNKS // 2, pair_body, 0)

    last = N_CHUNKS - 2
    pltpu.make_async_copy(ou0, out_slice(last), so0).wait()
    pltpu.make_async_copy(ou1, out_slice(last + 1), so1).wait()


def kernel(x):
    return _dn_quantize(x)


# speedup vs baseline: 2.4082x; 1.6241x over previous
import functools

import jax
import jax.numpy as jnp
import numpy as np
from jax import lax
from jax.experimental import pallas as pl
from jax.experimental.pallas import tpu as pltpu
from jax.experimental.pallas import tpu_sc as plsc

N_ROWS = 65536
N_COLS = 64
MAGIC = np.float32(12582912.0)

NC = 2
NS = 16
L = 16
NW = NC * NS
ROWS_PER_W = N_ROWS // NW
CH = 256
N_CHUNKS = ROWS_PER_W // CH
GROUPS = CH // L
N_CHAINS = 4

_mesh = plsc.VectorSubcoreMesh(core_axis_name="c", subcore_axis_name="s")


@functools.partial(
    pl.kernel,
    mesh=_mesh,
    out_type=jax.ShapeDtypeStruct((N_COLS, N_ROWS), jnp.float32),
    scratch_types=[
        pltpu.VMEM((N_COLS, CH), jnp.float32),
        pltpu.VMEM((N_COLS, CH), jnp.float32),
        pltpu.VMEM((N_COLS, CH), jnp.float32),
        pltpu.VMEM((N_COLS, CH), jnp.float32),
        pltpu.SemaphoreType.DMA,
        pltpu.SemaphoreType.DMA,
        pltpu.SemaphoreType.DMA,
        pltpu.SemaphoreType.DMA,
    ],
    compiler_params=pltpu.CompilerParams(needs_layout_passes=False),
)
def _dn_quantize(xt_hbm, out_hbm, in0, in1, ou0, ou1, si0, si1, so0, so1):
    wid = lax.axis_index("s") * NC + lax.axis_index("c")
    iota = lax.iota(jnp.int32, L)
    w_col0 = wid * ROWS_PER_W

    def in_slice(t):
        return xt_hbm.at[:, pl.ds(w_col0 + t * CH, CH)]

    def out_slice(t):
        return out_hbm.at[:, pl.ds(w_col0 + t * CH, CH)]

    def compute_chunk(in_buf, out_buf):
        @plsc.parallel_loop(0, GROUPS, unroll=2)
        def _grp(g):
            g0 = g * L
            posv = iota + g0
            m = [jnp.full((L,), -1.0, jnp.float32)] * N_CHAINS
            kb = [jnp.zeros((L,), jnp.int32)] * N_CHAINS
            sm = [jnp.zeros((L,), jnp.float32)] * N_CHAINS
            span = N_COLS // N_CHAINS
            for j in range(N_COLS):
                c = j // span
                v = in_buf[j, pl.ds(g0, L)]
                f = (v + MAGIC) - MAGIC
                out_buf[j, pl.ds(g0, L)] = f
                a = jnp.abs(v - f)
                p = a > m[c]
                m[c] = jnp.where(p, a, m[c])
                kb[c] = jnp.where(p, jnp.full((L,), j, jnp.int32), kb[c])
                sm[c] = sm[c] + f
            mm, kk, ss = m[0], kb[0], sm[0]
            for c in range(1, N_CHAINS):
                p = m[c] > mm
                mm = jnp.where(p, m[c], mm)
                kk = jnp.where(p, kb[c], kk)
                ss = ss + sm[c]
            odd = (ss.astype(jnp.int32) & 1) == 1
            vk = plsc.load_gather(in_buf, [kk, posv])
            fk = (vk + MAGIC) - MAGIC
            fix = jnp.where(vk - fk < 0, jnp.float32(-1.0), jnp.float32(1.0))
            plsc.addupdate_scatter(out_buf, [kk, posv], fix, mask=odd)

    def slot(u, t, in_buf, out_buf, in_sem, out_sem):
        @pl.when(u > 0)
        def _():
            pltpu.make_async_copy(out_buf, out_slice(t - 2), out_sem).wait()

        pltpu.make_async_copy(in_slice(t), in_buf, in_sem).wait()
        compute_chunk(in_buf, out_buf)
        pltpu.async_copy(out_buf, out_slice(t), out_sem)

        @pl.when(u < N_CHUNKS // 2 - 1)
        def _():
            pltpu.async_copy(in_slice(t + 2), in_buf, in_sem)

    pltpu.async_copy(in_slice(0), in0, si0)
    pltpu.async_copy(in_slice(1), in1, si1)

    def pair_body(u, carry):
        slot(u, 2 * u, in0, ou0, si0, so0)
        slot(u, 2 * u + 1, in1, ou1, si1, so1)
        return carry

    lax.fori_loop(0, N_CHUNKS // 2, pair_body, 0)

    last = N_CHUNKS - 2
    pltpu.make_async_copy(ou0, out_slice(last), so0).wait()
    pltpu.make_async_copy(ou1, out_slice(last + 1), so1).wait()


def kernel(x):
    return _dn_quantize(x.T).T


# speedup vs baseline: 2.5695x; 1.0670x over previous
import functools

import jax
import jax.numpy as jnp
import numpy as np
from jax import lax
from jax.experimental import pallas as pl
from jax.experimental.pallas import tpu as pltpu
from jax.experimental.pallas import tpu_sc as plsc

N_ROWS = 65536
N_COLS = 64
MAGIC = np.float32(12582912.0)

NC = 2
NS = 16
L = 16
NW = NC * NS
ROWS_PER_W = N_ROWS // NW
CH = 256
N_CHUNKS = ROWS_PER_W // CH
GROUPS = CH // L
N_CHAINS = 4

_mesh = plsc.VectorSubcoreMesh(core_axis_name="c", subcore_axis_name="s")


@functools.partial(
    pl.kernel,
    mesh=_mesh,
    out_type=jax.ShapeDtypeStruct((N_COLS, N_ROWS), jnp.float32),
    scratch_types=[
        pltpu.VMEM((N_COLS, CH), jnp.float32),
        pltpu.VMEM((N_COLS, CH), jnp.float32),
        pltpu.VMEM((N_COLS, CH), jnp.float32),
        pltpu.VMEM((N_COLS, CH), jnp.float32),
        pltpu.SemaphoreType.DMA,
        pltpu.SemaphoreType.DMA,
        pltpu.SemaphoreType.DMA,
        pltpu.SemaphoreType.DMA,
    ],
    compiler_params=pltpu.CompilerParams(needs_layout_passes=False),
)
def _dn_quantize(xt_hbm, out_hbm, in0, in1, ou0, ou1, si0, si1, so0, so1):
    wid = lax.axis_index("s") * NC + lax.axis_index("c")
    iota = lax.iota(jnp.int32, L)
    w_col0 = wid * ROWS_PER_W

    def in_slice(t):
        return xt_hbm.at[:, pl.ds(w_col0 + t * CH, CH)]

    def out_slice(t):
        return out_hbm.at[:, pl.ds(w_col0 + t * CH, CH)]

    def compute_chunk(in_buf, out_buf):
        @plsc.parallel_loop(0, GROUPS, unroll=4)
        def _grp(g):
            g0 = g * L
            posv = iota + g0
            m = [jnp.full((L,), -1.0, jnp.float32)] * N_CHAINS
            kb = [jnp.zeros((L,), jnp.int32)] * N_CHAINS
            sm = [jnp.zeros((L,), jnp.float32)] * N_CHAINS
            span = N_COLS // N_CHAINS
            for j in range(N_COLS):
                c = j // span
                v = in_buf[j, pl.ds(g0, L)]
                f = (v + MAGIC) - MAGIC
                out_buf[j, pl.ds(g0, L)] = f
                a = jnp.abs(v - f)
                p = a > m[c]
                m[c] = jnp.where(p, a, m[c])
                kb[c] = jnp.where(p, jnp.full((L,), j, jnp.int32), kb[c])
                sm[c] = sm[c] + f
            mm, kk, ss = m[0], kb[0], sm[0]
            for c in range(1, N_CHAINS):
                p = m[c] > mm
                mm = jnp.where(p, m[c], mm)
                kk = jnp.where(p, kb[c], kk)
                ss = ss + sm[c]
            odd = (ss.astype(jnp.int32) & 1) == 1
            vk = plsc.load_gather(in_buf, [kk, posv])
            fk = (vk + MAGIC) - MAGIC
            fix = jnp.where(vk - fk < 0, jnp.float32(-1.0), jnp.float32(1.0))
            plsc.addupdate_scatter(out_buf, [kk, posv], fix, mask=odd)

    def slot(u, t, in_buf, out_buf, in_sem, out_sem):
        @pl.when(u > 0)
        def _():
            pltpu.make_async_copy(out_buf, out_slice(t - 2), out_sem).wait()

        pltpu.make_async_copy(in_slice(t), in_buf, in_sem).wait()
        compute_chunk(in_buf, out_buf)
        pltpu.async_copy(out_buf, out_slice(t), out_sem)

        @pl.when(u < N_CHUNKS // 2 - 1)
        def _():
            pltpu.async_copy(in_slice(t + 2), in_buf, in_sem)

    pltpu.async_copy(in_slice(0), in0, si0)
    pltpu.async_copy(in_slice(1), in1, si1)

    def pair_body(u, carry):
        slot(u, 2 * u, in0, ou0, si0, so0)
        slot(u, 2 * u + 1, in1, ou1, si1, so1)
        return carry

    lax.fori_loop(0, N_CHUNKS // 2, pair_body, 0)

    last = N_CHUNKS - 2
    pltpu.make_async_copy(ou0, out_slice(last), so0).wait()
    pltpu.make_async_copy(ou1, out_slice(last + 1), so1).wait()


def kernel(x):
    return _dn_quantize(x.T).T
